# baseline (speedup 1.0000x reference)
import functools

import jax
import jax.numpy as jnp
from jax import lax
from jax.experimental import pallas as pl
from jax.experimental.pallas import tpu as pltpu
from jax.experimental.pallas import tpu_sc as plsc

R, C = 128, 32768
N = R * C
K = int(N * 10 / 100)
NC, NS, L = 2, 16, 16
NW = NC * NS
PER_TILE = N // NW
CH = 16384
NCH = PER_TILE // CH
UNROLL = 8
B1 = 2048
B2 = 2048

_mesh = plsc.VectorSubcoreMesh(core_axis_name="c", subcore_axis_name="s")
_sc_params = pltpu.CompilerParams(needs_layout_passes=False,
                                  use_tc_tiling_on_sc=False)


def _bce_body(x_ref, t_ref, o_ref):
    x = x_ref[...]
    t = t_ref[...]
    bce = jnp.maximum(x, 0.0) - x * t + jnp.log1p(jnp.exp(-jnp.abs(x)))
    o_ref[...] = bce.reshape(-1)


_bce_call = pl.pallas_call(
    _bce_body,
    grid=(16,),
    in_specs=[pl.BlockSpec((8, C), lambda i: (i, 0)),
              pl.BlockSpec((8, C), lambda i: (i, 0))],
    out_specs=pl.BlockSpec((8 * C, ), lambda i: (i, )),
    out_shape=jax.ShapeDtypeStruct((N, ), jnp.float32),
)


def _zero_hist(hist, nbins):
    zeros = jnp.zeros((L,), jnp.int32)

    def z(i, carry):
        for u in range(8):
            hist[i * 8 + u] = zeros
        return carry

    lax.fori_loop(0, nbins // 8, z, 0)


def _stream(hbm, tile_base, buf0, buf1, sem0, sem1, proc, carry):
    pltpu.async_copy(hbm.at[pl.ds(tile_base, CH)], buf0, sem0)
    pltpu.async_copy(hbm.at[pl.ds(tile_base + CH, CH)], buf1, sem1)

    def outer(g, c):
        base = tile_base + g * 2 * CH
        pltpu.make_async_copy(hbm.at[pl.ds(base, CH)], buf0, sem0).wait()
        c = proc(buf0, c)
        pltpu.async_copy(hbm.at[pl.ds(base + 2 * CH, CH)], buf0, sem0)
        pltpu.make_async_copy(hbm.at[pl.ds(base + CH, CH)], buf1, sem1).wait()
        c = proc(buf1, c)
        pltpu.async_copy(hbm.at[pl.ds(base + 3 * CH, CH)], buf1, sem1)
        return c

    carry = lax.fori_loop(0, NCH // 2 - 1, outer, carry)
    base = tile_base + (NCH - 2) * CH
    pltpu.make_async_copy(hbm.at[pl.ds(base, CH)], buf0, sem0).wait()
    carry = proc(buf0, carry)
    pltpu.make_async_copy(hbm.at[pl.ds(base + CH, CH)], buf1, sem1).wait()
    return proc(buf1, carry)


def _lane_reduce(hist, hist_red, nbins):
    iota = lax.iota(jnp.int32, L)

    def grp(g, carry):
        bin_ids = g * L + iota
        w = jnp.zeros((L,), jnp.int32)
        for l in range(L):
            lane = jnp.full((L,), l, jnp.int32)
            w = w + plsc.load_gather(hist, [bin_ids, lane])
        hist_red[pl.ds(g * L, L)] = w
        return carry

    lax.fori_loop(0, nbins // L, grp, 0)


def _lane_reduce2(hist_a, hist_b, hist_red, nbins):
    iota = lax.iota(jnp.int32, L)

    def grp(g, carry):
        bin_ids = g * L + iota
        w = jnp.zeros((L,), jnp.int32)
        for l in range(L):
            lane = jnp.full((L,), l, jnp.int32)
            w = w + plsc.load_gather(hist_a, [bin_ids, lane])
            w = w + plsc.load_gather(hist_b, [bin_ids, lane])
        hist_red[pl.ds(g * L, L)] = w
        return carry

    lax.fori_loop(0, nbins // L, grp, 0)


@functools.partial(
    pl.kernel,
    out_type=jax.ShapeDtypeStruct((NW, B1), jnp.int32),
    mesh=_mesh,
    compiler_params=_sc_params,
    scratch_types=[
        pltpu.VMEM((CH,), jnp.float32),
        pltpu.VMEM((CH,), jnp.float32),
        pltpu.VMEM((B1, L), jnp.int32),
        pltpu.VMEM((B1, L), jnp.int32),
        pltpu.VMEM((B1,), jnp.int32),
        pltpu.SemaphoreType.DMA,
        pltpu.SemaphoreType.DMA,
    ],
)
def _sc_pass1(bce_hbm, h1_out, buf0, buf1, hist, histb, hist_red, sem0, sem1):
    wid = lax.axis_index("c") * NS + lax.axis_index("s")
    _zero_hist(hist, B1)
    _zero_hist(histb, B1)
    lanes = lax.iota(jnp.int32, L)
    ones = jnp.ones((L,), jnp.int32)

    def proc(buf, carry):
        def vec(vi, c2):
            base = vi * (L * UNROLL)
            idxs = []
            for u in range(UNROLL):
                v = buf[pl.ds(base + u * L, L)]
                bits = plsc.bitcast(v, jnp.int32)
                idxs.append(lax.shift_right_logical(bits, 20))
            for u, idx in enumerate(idxs):
                plsc.addupdate_scatter(hist if u % 2 == 0 else histb,
                                       [idx, lanes], ones)
            return c2

        return lax.fori_loop(0, CH // (L * UNROLL), vec, carry)

    _stream(bce_hbm, wid * PER_TILE, buf0, buf1, sem0, sem1, proc, 0)
    _lane_reduce2(hist, histb, hist_red, B1)
    pltpu.sync_copy(hist_red, h1_out.at[wid])


@functools.partial(
    pl.kernel,
    out_type=(jax.ShapeDtypeStruct((NW, B2), jnp.int32),
              jax.ShapeDtypeStruct((NW, L), jnp.float32)),
    mesh=_mesh,
    compiler_params=_sc_params,
    scratch_types=[
        pltpu.VMEM((CH,), jnp.float32),
        pltpu.VMEM((CH,), jnp.float32),
        pltpu.VMEM((B2, L), jnp.int32),
        pltpu.VMEM((B2, L), jnp.int32),
        pltpu.VMEM((B2,), jnp.int32),
        pltpu.VMEM((L,), jnp.int32),
        pltpu.VMEM((L,), jnp.float32),
        pltpu.SemaphoreType.DMA,
        pltpu.SemaphoreType.DMA,
    ],
)
def _sc_pass2(bce_hbm, params_hbm, h2_out, sums_out, buf0, buf1,
              hist, histb, hist_red, pbuf, sbuf, sem0, sem1):
    wid = lax.axis_index("c") * NS + lax.axis_index("s")
    pltpu.sync_copy(params_hbm.at[pl.ds(0, L)], pbuf)
    a_vec = pbuf[...]
    _zero_hist(hist, B2)
    _zero_hist(histb, B2)
    lanes = lax.iota(jnp.int32, L)
    ones = jnp.ones((L,), jnp.int32)
    zero = jnp.zeros((L,), jnp.float32)

    def proc(buf, accs):
        def vec(vi, accs2):
            base = vi * (L * UNROLL)
            vals, tops, mids = [], [], []
            for u in range(UNROLL):
                v = buf[pl.ds(base + u * L, L)]
                bits = plsc.bitcast(v, jnp.int32)
                vals.append(v)
                tops.append(lax.shift_right_logical(bits, 20))
                mids.append(
                    jnp.bitwise_and(lax.shift_right_logical(bits, 9), B2 - 1))
            for u in range(UNROLL):
                plsc.addupdate_scatter(hist if u % 2 == 0 else histb,
                                       [mids[u], lanes], ones,
                                       mask=tops[u] == a_vec)
            return tuple(
                accs2[u] + jnp.where(tops[u] > a_vec, vals[u], 0.0)
                for u in range(UNROLL))

        return lax.fori_loop(0, CH // (L * UNROLL), vec, accs)

    accs = _stream(bce_hbm, wid * PER_TILE, buf0, buf1, sem0, sem1, proc,
                   (zero,) * UNROLL)
    accs = list(accs)
    while len(accs) > 1:
        accs = [a + b for a, b in zip(accs[::2], accs[1::2])]
    sbuf[...] = accs[0]
    _lane_reduce2(hist, histb, hist_red, B2)
    pltpu.sync_copy(hist_red, h2_out.at[wid])
    pltpu.sync_copy(sbuf, sums_out.at[wid])


def _suffix_scan(t16x128):
    s = t16x128
    for sh in (1, 2, 4, 8, 16, 32, 64):
        s = s + jnp.concatenate(
            [s[:, sh:], jnp.zeros((16, sh), jnp.float32)], axis=1)
    rows = s[:, 0:1]
    gi = lax.broadcasted_iota(jnp.int32, (16, 16), 0)
    gj = lax.broadcasted_iota(jnp.int32, (16, 16), 1)
    below = jnp.sum(jnp.where(gi > gj, rows, 0.0), axis=0)
    return s + below[:, None]


def _flat_iota_i():
    return (lax.broadcasted_iota(jnp.int32, (16, 128), 0) * 128
            + lax.broadcasted_iota(jnp.int32, (16, 128), 1))


def _flat_iota():
    return _flat_iota_i().astype(jnp.float32)


def _scan1_body(h1_ref, p_ref):
    t = jnp.sum(h1_ref[...].astype(jnp.float32), axis=0)
    t = t.reshape(16, 128)
    s = _suffix_scan(t)
    msk = (s >= float(K)).astype(jnp.float32)
    a_star = jnp.sum(msk) - 1.0
    above = jnp.sum(jnp.where(_flat_iota() > a_star, t, 0.0))
    kp = float(K) - above
    ri = lax.broadcasted_iota(jnp.int32, (8, 128), 0)
    p = jnp.where(ri == 0, a_star.astype(jnp.int32),
                  jnp.where(ri == 1, kp.astype(jnp.int32), 0))
    p_ref[...] = p


_scan1_call = pl.pallas_call(
    _scan1_body,
    out_shape=jax.ShapeDtypeStruct((8, 128), jnp.int32),
)


def _final_body(h2_ref, sums_ref, p_ref, o_ref):
    u = jnp.sum(h2_ref[...].astype(jnp.float32), axis=0)
    u = u.reshape(16, 128)
    a_star = p_ref[0, 0]
    kp = p_ref[1, 0].astype(jnp.float32)
    s = _suffix_scan(u)
    msk = (s >= kp).astype(jnp.float32)
    b_star = jnp.sum(msk) - 1.0
    flat = _flat_iota()
    above2 = jnp.sum(jnp.where(flat > b_star, u, 0.0))
    k2 = kp - above2
    flat_i = _flat_iota_i()
    vbits = jnp.bitwise_or(
        jnp.bitwise_or(lax.shift_left(a_star, 20), lax.shift_left(flat_i, 9)),
        256)
    vhat = lax.bitcast_convert_type(vbits, jnp.float32)
    sum_mid = jnp.sum(jnp.where(flat > b_star, u * vhat, 0.0))
    v_b = jnp.sum(jnp.where(flat == b_star, vhat, 0.0))
    total = jnp.sum(sums_ref[...]) + sum_mid + k2 * v_b
    o_ref[...] = jnp.broadcast_to(total / float(K), (1, 1))


_final_call = pl.pallas_call(
    _final_body,
    out_shape=jax.ShapeDtypeStruct((1, 1), jnp.float32),
)


def kernel(inputs, targets):
    bce_flat = _bce_call(inputs, targets)
    h1 = _sc_pass1(bce_flat)
    params = _scan1_call(h1)
    h2, sums = _sc_pass2(bce_flat, params.reshape(-1))
    out = _final_call(h2, sums, params)
    return out.reshape(())


# speedup vs baseline: 1.2577x; 1.2577x over previous
import functools

import jax
import jax.numpy as jnp
from jax import lax
from jax.experimental import pallas as pl
from jax.experimental.pallas import tpu as pltpu
from jax.experimental.pallas import tpu_sc as plsc

R, C = 128, 32768
N = R * C
K = int(N * 10 / 100)
NC, NS, L = 2, 16, 16
NW = NC * NS
PER_TILE = N // NW
CH = 16384
NCH = PER_TILE // CH
UNROLL = 16
B1 = 2048
B2 = 2048

_mesh = plsc.VectorSubcoreMesh(core_axis_name="c", subcore_axis_name="s")
_sc_params = pltpu.CompilerParams(needs_layout_passes=False,
                                  use_tc_tiling_on_sc=False)


def _bce_body(x_ref, t_ref, o_ref):
    x = x_ref[...]
    t = t_ref[...]
    bce = jnp.maximum(x, 0.0) - x * t + jnp.log1p(jnp.exp(-jnp.abs(x)))
    o_ref[...] = bce.reshape(-1)


_bce_call = pl.pallas_call(
    _bce_body,
    grid=(16,),
    in_specs=[pl.BlockSpec((8, C), lambda i: (i, 0)),
              pl.BlockSpec((8, C), lambda i: (i, 0))],
    out_specs=pl.BlockSpec((8 * C, ), lambda i: (i, )),
    out_shape=jax.ShapeDtypeStruct((N, ), jnp.float32),
)


def _zero_hist(hist, nbins):
    zeros = jnp.zeros((L,), jnp.int32)

    def z(i, carry):
        for u in range(8):
            hist[i * 8 + u] = zeros
        return carry

    lax.fori_loop(0, nbins // 8, z, 0)


def _stream(hbm, tile_base, buf0, buf1, sem0, sem1, proc, carry):
    pltpu.async_copy(hbm.at[pl.ds(tile_base, CH)], buf0, sem0)
    pltpu.async_copy(hbm.at[pl.ds(tile_base + CH, CH)], buf1, sem1)

    def outer(g, c):
        base = tile_base + g * 2 * CH
        pltpu.make_async_copy(hbm.at[pl.ds(base, CH)], buf0, sem0).wait()
        c = proc(buf0, c)
        pltpu.async_copy(hbm.at[pl.ds(base + 2 * CH, CH)], buf0, sem0)
        pltpu.make_async_copy(hbm.at[pl.ds(base + CH, CH)], buf1, sem1).wait()
        c = proc(buf1, c)
        pltpu.async_copy(hbm.at[pl.ds(base + 3 * CH, CH)], buf1, sem1)
        return c

    carry = lax.fori_loop(0, NCH // 2 - 1, outer, carry)
    base = tile_base + (NCH - 2) * CH
    pltpu.make_async_copy(hbm.at[pl.ds(base, CH)], buf0, sem0).wait()
    carry = proc(buf0, carry)
    pltpu.make_async_copy(hbm.at[pl.ds(base + CH, CH)], buf1, sem1).wait()
    return proc(buf1, carry)


def _lane_reduce(hist, hist_red, nbins):
    iota = lax.iota(jnp.int32, L)

    def grp(g, carry):
        bin_ids = g * L + iota
        w = jnp.zeros((L,), jnp.int32)
        for l in range(L):
            lane = jnp.full((L,), l, jnp.int32)
            w = w + plsc.load_gather(hist, [bin_ids, lane])
        hist_red[pl.ds(g * L, L)] = w
        return carry

    lax.fori_loop(0, nbins // L, grp, 0)


@functools.partial(
    pl.kernel,
    out_type=jax.ShapeDtypeStruct((NW, B1), jnp.int32),
    mesh=_mesh,
    compiler_params=_sc_params,
    scratch_types=[
        pltpu.VMEM((CH,), jnp.float32),
        pltpu.VMEM((CH,), jnp.float32),
        pltpu.VMEM((B1, L), jnp.int32),
        pltpu.VMEM((B1,), jnp.int32),
        pltpu.SemaphoreType.DMA,
        pltpu.SemaphoreType.DMA,
    ],
)
def _sc_pass1(bce_hbm, h1_out, buf0, buf1, hist, hist_red, sem0, sem1):
    wid = lax.axis_index("c") * NS + lax.axis_index("s")
    _zero_hist(hist, B1)
    lanes = lax.iota(jnp.int32, L)
    ones = jnp.ones((L,), jnp.int32)

    def proc(buf, carry):
        def vec(vi, c2):
            base = vi * (L * UNROLL)
            idxs = []
            for u in range(UNROLL):
                v = buf[pl.ds(base + u * L, L)]
                bits = plsc.bitcast(v, jnp.int32)
                idxs.append(lax.shift_right_logical(bits, 20))
            for idx in idxs:
                plsc.addupdate_scatter(hist, [idx, lanes], ones)
            return c2

        return lax.fori_loop(0, CH // (L * UNROLL), vec, carry)

    _stream(bce_hbm, wid * PER_TILE, buf0, buf1, sem0, sem1, proc, 0)
    _lane_reduce(hist, hist_red, B1)
    pltpu.sync_copy(hist_red, h1_out.at[wid])


@functools.partial(
    pl.kernel,
    out_type=(jax.ShapeDtypeStruct((NW, B2), jnp.int32),
              jax.ShapeDtypeStruct((NW, L), jnp.float32)),
    mesh=_mesh,
    compiler_params=_sc_params,
    scratch_types=[
        pltpu.VMEM((CH,), jnp.float32),
        pltpu.VMEM((CH,), jnp.float32),
        pltpu.VMEM((B2, L), jnp.int32),
        pltpu.VMEM((B2,), jnp.int32),
        pltpu.VMEM((L,), jnp.int32),
        pltpu.VMEM((L,), jnp.float32),
        pltpu.SemaphoreType.DMA,
        pltpu.SemaphoreType.DMA,
    ],
)
def _sc_pass2(bce_hbm, params_hbm, h2_out, sums_out, buf0, buf1,
              hist, hist_red, pbuf, sbuf, sem0, sem1):
    wid = lax.axis_index("c") * NS + lax.axis_index("s")
    pltpu.sync_copy(params_hbm.at[pl.ds(0, L)], pbuf)
    a_vec = pbuf[...]
    _zero_hist(hist, B2)
    lanes = lax.iota(jnp.int32, L)
    ones = jnp.ones((L,), jnp.int32)
    zero = jnp.zeros((L,), jnp.float32)

    def proc(buf, accs):
        def vec(vi, accs2):
            base = vi * (L * UNROLL)
            vals, tops, mids = [], [], []
            for u in range(UNROLL):
                v = buf[pl.ds(base + u * L, L)]
                bits = plsc.bitcast(v, jnp.int32)
                vals.append(v)
                tops.append(lax.shift_right_logical(bits, 20))
                mids.append(
                    jnp.bitwise_and(lax.shift_right_logical(bits, 9), B2 - 1))
            for u in range(UNROLL):
                plsc.addupdate_scatter(hist, [mids[u], lanes], ones,
                                       mask=tops[u] == a_vec)
            return tuple(
                accs2[u] + jnp.where(tops[u] > a_vec, vals[u], 0.0)
                for u in range(UNROLL))

        return lax.fori_loop(0, CH // (L * UNROLL), vec, accs)

    accs = _stream(bce_hbm, wid * PER_TILE, buf0, buf1, sem0, sem1, proc,
                   (zero,) * UNROLL)
    accs = list(accs)
    while len(accs) > 1:
        accs = [a + b for a, b in zip(accs[::2], accs[1::2])]
    sbuf[...] = accs[0]
    _lane_reduce(hist, hist_red, B2)
    pltpu.sync_copy(hist_red, h2_out.at[wid])
    pltpu.sync_copy(sbuf, sums_out.at[wid])


def _suffix_scan(t16x128):
    s = t16x128
    for sh in (1, 2, 4, 8, 16, 32, 64):
        s = s + jnp.concatenate(
            [s[:, sh:], jnp.zeros((16, sh), jnp.float32)], axis=1)
    rows = s[:, 0:1]
    gi = lax.broadcasted_iota(jnp.int32, (16, 16), 0)
    gj = lax.broadcasted_iota(jnp.int32, (16, 16), 1)
    below = jnp.sum(jnp.where(gi > gj, rows, 0.0), axis=0)
    return s + below[:, None]


def _flat_iota_i():
    return (lax.broadcasted_iota(jnp.int32, (16, 128), 0) * 128
            + lax.broadcasted_iota(jnp.int32, (16, 128), 1))


def _flat_iota():
    return _flat_iota_i().astype(jnp.float32)


def _scan1_body(h1_ref, p_ref):
    t = jnp.sum(h1_ref[...].astype(jnp.float32), axis=0)
    t = t.reshape(16, 128)
    s = _suffix_scan(t)
    msk = (s >= float(K)).astype(jnp.float32)
    a_star = jnp.sum(msk) - 1.0
    above = jnp.sum(jnp.where(_flat_iota() > a_star, t, 0.0))
    kp = float(K) - above
    ri = lax.broadcasted_iota(jnp.int32, (8, 128), 0)
    p = jnp.where(ri == 0, a_star.astype(jnp.int32),
                  jnp.where(ri == 1, kp.astype(jnp.int32), 0))
    p_ref[...] = p


_scan1_call = pl.pallas_call(
    _scan1_body,
    out_shape=jax.ShapeDtypeStruct((8, 128), jnp.int32),
)


def _final_body(h2_ref, sums_ref, p_ref, o_ref):
    u = jnp.sum(h2_ref[...].astype(jnp.float32), axis=0)
    u = u.reshape(16, 128)
    a_star = p_ref[0, 0]
    kp = p_ref[1, 0].astype(jnp.float32)
    s = _suffix_scan(u)
    msk = (s >= kp).astype(jnp.float32)
    b_star = jnp.sum(msk) - 1.0
    flat = _flat_iota()
    above2 = jnp.sum(jnp.where(flat > b_star, u, 0.0))
    k2 = kp - above2
    flat_i = _flat_iota_i()
    vbits = jnp.bitwise_or(
        jnp.bitwise_or(lax.shift_left(a_star, 20), lax.shift_left(flat_i, 9)),
        256)
    vhat = lax.bitcast_convert_type(vbits, jnp.float32)
    sum_mid = jnp.sum(jnp.where(flat > b_star, u * vhat, 0.0))
    v_b = jnp.sum(jnp.where(flat == b_star, vhat, 0.0))
    total = jnp.sum(sums_ref[...]) + sum_mid + k2 * v_b
    o_ref[...] = jnp.broadcast_to(total / float(K), (1, 1))


_final_call = pl.pallas_call(
    _final_body,
    out_shape=jax.ShapeDtypeStruct((1, 1), jnp.float32),
)


def kernel(inputs, targets):
    bce_flat = _bce_call(inputs, targets)
    h1 = _sc_pass1(bce_flat)
    params = _scan1_call(h1)
    h2, sums = _sc_pass2(bce_flat, params.reshape(-1))
    out = _final_call(h2, sums, params)
    return out.reshape(())


# speedup vs baseline: 1.2963x; 1.0307x over previous
import functools

import jax
import jax.numpy as jnp
from jax import lax
from jax.experimental import pallas as pl
from jax.experimental.pallas import tpu as pltpu
from jax.experimental.pallas import tpu_sc as plsc

R, C = 128, 32768
N = R * C
K = int(N * 10 / 100)
NC, NS, L = 2, 16, 16
NW = NC * NS
PER_TILE = N // NW
CH = 16384
NCH = PER_TILE // CH
UNROLL = 16
B1 = 2048
B2 = 2048

_mesh = plsc.VectorSubcoreMesh(core_axis_name="c", subcore_axis_name="s")
_sc_params = pltpu.CompilerParams(needs_layout_passes=False,
                                  use_tc_tiling_on_sc=False)


def _bce_body(x_ref, t_ref, o_ref):
    x = x_ref[...]
    t = t_ref[...]
    bce = jnp.maximum(x, 0.0) - x * t + jnp.log1p(jnp.exp(-jnp.abs(x)))
    o_ref[...] = bce.reshape(-1)


_bce_call = pl.pallas_call(
    _bce_body,
    grid=(16,),
    in_specs=[pl.BlockSpec((8, C), lambda i: (i, 0)),
              pl.BlockSpec((8, C), lambda i: (i, 0))],
    out_specs=pl.BlockSpec((8 * C, ), lambda i: (i, )),
    out_shape=jax.ShapeDtypeStruct((N, ), jnp.float32),
)


def _zero_hist(hist, nbins):
    zeros = jnp.zeros((L,), jnp.int32)

    def z(i, carry):
        for u in range(8):
            hist[i * 8 + u] = zeros
        return carry

    lax.fori_loop(0, nbins // 8, z, 0)


def _stream_prime(hbm, tile_base, buf0, buf1, sem0, sem1):
    pltpu.async_copy(hbm.at[pl.ds(tile_base, CH)], buf0, sem0)
    pltpu.async_copy(hbm.at[pl.ds(tile_base + CH, CH)], buf1, sem1)


def _stream(hbm, tile_base, buf0, buf1, sem0, sem1, proc, carry):
    def outer(g, c):
        base = tile_base + g * 2 * CH
        pltpu.make_async_copy(hbm.at[pl.ds(base, CH)], buf0, sem0).wait()
        c = proc(buf0, c)
        pltpu.async_copy(hbm.at[pl.ds(base + 2 * CH, CH)], buf0, sem0)
        pltpu.make_async_copy(hbm.at[pl.ds(base + CH, CH)], buf1, sem1).wait()
        c = proc(buf1, c)
        pltpu.async_copy(hbm.at[pl.ds(base + 3 * CH, CH)], buf1, sem1)
        return c

    carry = lax.fori_loop(0, NCH // 2 - 1, outer, carry)
    base = tile_base + (NCH - 2) * CH
    pltpu.make_async_copy(hbm.at[pl.ds(base, CH)], buf0, sem0).wait()
    carry = proc(buf0, carry)
    pltpu.make_async_copy(hbm.at[pl.ds(base + CH, CH)], buf1, sem1).wait()
    return proc(buf1, carry)


def _lane_reduce(hist, hist_red, nbins):
    iota = lax.iota(jnp.int32, L)

    def grp(g, carry):
        bin_ids = g * L + iota
        w = jnp.zeros((L,), jnp.int32)
        for l in range(L):
            lane = jnp.full((L,), l, jnp.int32)
            w = w + plsc.load_gather(hist, [bin_ids, lane])
        hist_red[pl.ds(g * L, L)] = w
        return carry

    lax.fori_loop(0, nbins // L, grp, 0)


@functools.partial(
    pl.kernel,
    out_type=jax.ShapeDtypeStruct((NW, B1), jnp.int32),
    mesh=_mesh,
    compiler_params=_sc_params,
    scratch_types=[
        pltpu.VMEM((CH,), jnp.float32),
        pltpu.VMEM((CH,), jnp.float32),
        pltpu.VMEM((B1, L), jnp.int32),
        pltpu.VMEM((B1,), jnp.int32),
        pltpu.SemaphoreType.DMA,
        pltpu.SemaphoreType.DMA,
    ],
)
def _sc_pass1(bce_hbm, h1_out, buf0, buf1, hist, hist_red, sem0, sem1):
    wid = lax.axis_index("c") * NS + lax.axis_index("s")
    _stream_prime(bce_hbm, wid * PER_TILE, buf0, buf1, sem0, sem1)
    _zero_hist(hist, B1)
    lanes = lax.iota(jnp.int32, L)
    ones = jnp.ones((L,), jnp.int32)

    def proc(buf, carry):
        def vec(vi, c2):
            base = vi * (L * UNROLL)
            idxs = []
            for u in range(UNROLL):
                v = buf[pl.ds(base + u * L, L)]
                bits = plsc.bitcast(v, jnp.int32)
                idxs.append(lax.shift_right_logical(bits, 20))
            for idx in idxs:
                plsc.addupdate_scatter(hist, [idx, lanes], ones)
            return c2

        return lax.fori_loop(0, CH // (L * UNROLL), vec, carry)

    _stream(bce_hbm, wid * PER_TILE, buf0, buf1, sem0, sem1, proc, 0)
    _lane_reduce(hist, hist_red, B1)
    pltpu.sync_copy(hist_red, h1_out.at[wid])


@functools.partial(
    pl.kernel,
    out_type=(jax.ShapeDtypeStruct((NW, B2), jnp.int32),
              jax.ShapeDtypeStruct((NW, L), jnp.float32)),
    mesh=_mesh,
    compiler_params=_sc_params,
    scratch_types=[
        pltpu.VMEM((CH,), jnp.float32),
        pltpu.VMEM((CH,), jnp.float32),
        pltpu.VMEM((B2, L), jnp.int32),
        pltpu.VMEM((B2,), jnp.int32),
        pltpu.VMEM((L,), jnp.int32),
        pltpu.VMEM((L,), jnp.float32),
        pltpu.SemaphoreType.DMA,
        pltpu.SemaphoreType.DMA,
    ],
)
def _sc_pass2(bce_hbm, params_hbm, h2_out, sums_out, buf0, buf1,
              hist, hist_red, pbuf, sbuf, sem0, sem1):
    wid = lax.axis_index("c") * NS + lax.axis_index("s")
    _stream_prime(bce_hbm, wid * PER_TILE, buf0, buf1, sem0, sem1)
    pltpu.sync_copy(params_hbm.at[pl.ds(0, L)], pbuf)
    a_vec = pbuf[...]
    _zero_hist(hist, B2)
    lanes = lax.iota(jnp.int32, L)
    ones = jnp.ones((L,), jnp.int32)
    zero = jnp.zeros((L,), jnp.float32)

    def proc(buf, accs):
        def vec(vi, accs2):
            base = vi * (L * UNROLL)
            vals, tops, mids = [], [], []
            for u in range(UNROLL):
                v = buf[pl.ds(base + u * L, L)]
                bits = plsc.bitcast(v, jnp.int32)
                vals.append(v)
                tops.append(lax.shift_right_logical(bits, 20))
                mids.append(
                    jnp.bitwise_and(lax.shift_right_logical(bits, 9), B2 - 1))
            for u in range(UNROLL):
                plsc.addupdate_scatter(hist, [mids[u], lanes], ones,
                                       mask=tops[u] == a_vec)
            return tuple(
                accs2[u] + jnp.where(tops[u] > a_vec, vals[u], 0.0)
                for u in range(UNROLL))

        return lax.fori_loop(0, CH // (L * UNROLL), vec, accs)

    accs = _stream(bce_hbm, wid * PER_TILE, buf0, buf1, sem0, sem1, proc,
                   (zero,) * UNROLL)
    accs = list(accs)
    while len(accs) > 1:
        accs = [a + b for a, b in zip(accs[::2], accs[1::2])]
    sbuf[...] = accs[0]
    _lane_reduce(hist, hist_red, B2)
    pltpu.sync_copy(hist_red, h2_out.at[wid])
    pltpu.sync_copy(sbuf, sums_out.at[wid])


def _suffix_scan(t16x128):
    s = t16x128
    for sh in (1, 2, 4, 8, 16, 32, 64):
        s = s + jnp.concatenate(
            [s[:, sh:], jnp.zeros((16, sh), jnp.float32)], axis=1)
    rows = s[:, 0:1]
    gi = lax.broadcasted_iota(jnp.int32, (16, 16), 0)
    gj = lax.broadcasted_iota(jnp.int32, (16, 16), 1)
    below = jnp.sum(jnp.where(gi > gj, rows, 0.0), axis=0)
    return s + below[:, None]


def _flat_iota_i():
    return (lax.broadcasted_iota(jnp.int32, (16, 128), 0) * 128
            + lax.broadcasted_iota(jnp.int32, (16, 128), 1))


def _flat_iota():
    return _flat_iota_i().astype(jnp.float32)


def _scan1_body(h1_ref, p_ref):
    t = jnp.sum(h1_ref[...].astype(jnp.float32), axis=0)
    t = t.reshape(16, 128)
    s = _suffix_scan(t)
    msk = (s >= float(K)).astype(jnp.float32)
    a_star = jnp.sum(msk) - 1.0
    above = jnp.sum(jnp.where(_flat_iota() > a_star, t, 0.0))
    kp = float(K) - above
    ri = lax.broadcasted_iota(jnp.int32, (8, 128), 0)
    p = jnp.where(ri == 0, a_star.astype(jnp.int32),
                  jnp.where(ri == 1, kp.astype(jnp.int32), 0))
    p_ref[...] = p


_scan1_call = pl.pallas_call(
    _scan1_body,
    out_shape=jax.ShapeDtypeStruct((8, 128), jnp.int32),
)


def _final_body(h2_ref, sums_ref, p_ref, o_ref):
    u = jnp.sum(h2_ref[...].astype(jnp.float32), axis=0)
    u = u.reshape(16, 128)
    a_star = p_ref[0, 0]
    kp = p_ref[1, 0].astype(jnp.float32)
    s = _suffix_scan(u)
    msk = (s >= kp).astype(jnp.float32)
    b_star = jnp.sum(msk) - 1.0
    flat = _flat_iota()
    above2 = jnp.sum(jnp.where(flat > b_star, u, 0.0))
    k2 = kp - above2
    flat_i = _flat_iota_i()
    vbits = jnp.bitwise_or(
        jnp.bitwise_or(lax.shift_left(a_star, 20), lax.shift_left(flat_i, 9)),
        256)
    vhat = lax.bitcast_convert_type(vbits, jnp.float32)
    sum_mid = jnp.sum(jnp.where(flat > b_star, u * vhat, 0.0))
    v_b = jnp.sum(jnp.where(flat == b_star, vhat, 0.0))
    total = jnp.sum(sums_ref[...]) + sum_mid + k2 * v_b
    o_ref[...] = jnp.broadcast_to(total / float(K), (1, 1))


_final_call = pl.pallas_call(
    _final_body,
    out_shape=jax.ShapeDtypeStruct((1, 1), jnp.float32),
)


def kernel(inputs, targets):
    bce_flat = _bce_call(inputs, targets)
    h1 = _sc_pass1(bce_flat)
    params = _scan1_call(h1)
    h2, sums = _sc_pass2(bce_flat, params.reshape(-1))
    out = _final_call(h2, sums, params)
    return out.reshape(())
